# baseline (speedup 1.0000x reference)
import jax
import jax.numpy as jnp
from jax.experimental import pallas as pl
from jax.experimental.pallas import tpu as pltpu

TOK = 8192
HIDDEN = 768
OUT = 768
CTX = 64
E = 8
HALF = HIDDEN // 2
TILE = 256


def _moe_tile_kernel(x_ref, c_ref, gw_ref, gb_ref, w1h_ref, w1c_ref,
                     b1_ref, w2_ref, b2_ref, out_ref, probs_ref):
    x = x_ref[...]
    c = c_ref[...]

    logits = jax.lax.dot_general(
        x, gw_ref[...], (((1,), (0,)), ((), ())),
        precision=jax.lax.Precision.DEFAULT,
        preferred_element_type=jnp.float32) + gb_ref[...]
    m = jnp.max(logits, axis=-1, keepdims=True)
    ex = jnp.exp(logits - m)
    probs = ex / jnp.sum(ex, axis=-1, keepdims=True)
    probs_ref[...] = probs

    idx = jax.lax.broadcasted_iota(jnp.int32, probs.shape, 1)
    p1 = jnp.max(probs, axis=-1, keepdims=True)
    i1 = jnp.min(jnp.where(probs == p1, idx, E), axis=-1, keepdims=True)
    oh1 = (idx == i1)
    masked = jnp.where(oh1, -jnp.inf, probs)
    p2 = jnp.max(masked, axis=-1, keepdims=True)
    i2 = jnp.min(jnp.where(masked == p2, idx, E), axis=-1, keepdims=True)
    oh2 = (idx == i2)
    denom = p1 + p2 + 1e-8
    coef = (jnp.where(oh1, probs, 0.0) + jnp.where(oh2, probs, 0.0)) / denom

    acc = jax.lax.dot_general(
        coef, b2_ref[...], (((1,), (0,)), ((), ())),
        precision=jax.lax.Precision.HIGHEST,
        preferred_element_type=jnp.float32)
    for e in range(E):
        h = (jax.lax.dot_general(
                x, w1h_ref[e], (((1,), (0,)), ((), ())),
                precision=jax.lax.Precision.HIGHEST,
                preferred_element_type=jnp.float32)
             + jax.lax.dot_general(
                c, w1c_ref[e], (((1,), (0,)), ((), ())),
                precision=jax.lax.Precision.HIGHEST,
                preferred_element_type=jnp.float32)
             + b1_ref[e][None, :])
        h = jnp.maximum(h, 0.0) * coef[:, e:e + 1]
        acc = acc + jax.lax.dot_general(
            h, w2_ref[e], (((1,), (0,)), ((), ())),
            precision=jax.lax.Precision.HIGHEST,
            preferred_element_type=jnp.float32)
    out_ref[...] = acc


def kernel(hidden_state, context, gate_W, gate_b, W1, b1, W2, b2):
    w1h = W1[:, :HIDDEN, :]
    w1c = W1[:, HIDDEN:, :]
    gb = gate_b.reshape(1, E)

    grid = (TOK // TILE,)
    out_shapes = (
        jax.ShapeDtypeStruct((TOK, OUT), jnp.float32),
        jax.ShapeDtypeStruct((TOK, E), jnp.float32),
    )
    full = lambda *dims: pl.BlockSpec(dims, lambda i: (0,) * len(dims))
    final_out, gate_probs = pl.pallas_call(
        _moe_tile_kernel,
        grid=grid,
        in_specs=[
            pl.BlockSpec((TILE, HIDDEN), lambda i: (i, 0)),
            pl.BlockSpec((TILE, CTX), lambda i: (i, 0)),
            full(HIDDEN, E),
            full(1, E),
            full(E, HIDDEN, HALF),
            full(E, CTX, HALF),
            full(E, HALF),
            full(E, HALF, OUT),
            full(E, OUT),
        ],
        out_specs=(
            pl.BlockSpec((TILE, OUT), lambda i: (i, 0)),
            pl.BlockSpec((TILE, E), lambda i: (i, 0)),
        ),
        out_shape=out_shapes,
        compiler_params=pltpu.CompilerParams(
            dimension_semantics=("arbitrary",),
        ),
    )(hidden_state, context, gate_W, gb, w1h, w1c, b1, W2, b2)
    return (final_out, gate_probs)


# speedup vs baseline: 3.7148x; 3.7148x over previous
import jax
import jax.numpy as jnp
from jax.experimental import pallas as pl
from jax.experimental.pallas import tpu as pltpu

TOK = 8192
HIDDEN = 768
OUT = 768
CTX = 64
E = 8
HALF = HIDDEN // 2
TILE = 256


def _moe_tile_kernel(x_ref, c_ref, gw_ref, gb_ref, w1h_ref, w1c_ref,
                     b1_ref, w2_ref, b2_ref, out_ref, probs_ref):
    x = x_ref[...]
    c = c_ref[...]

    logits = jax.lax.dot_general(
        x, gw_ref[...], (((1,), (0,)), ((), ())),
        precision=jax.lax.Precision.DEFAULT,
        preferred_element_type=jnp.float32) + gb_ref[...]
    m = jnp.max(logits, axis=-1, keepdims=True)
    ex = jnp.exp(logits - m)
    probs = ex / jnp.sum(ex, axis=-1, keepdims=True)
    probs_ref[...] = probs

    idx = jax.lax.broadcasted_iota(jnp.int32, probs.shape, 1)
    p1 = jnp.max(probs, axis=-1, keepdims=True)
    i1 = jnp.min(jnp.where(probs == p1, idx, E), axis=-1, keepdims=True)
    oh1 = (idx == i1)
    masked = jnp.where(oh1, -jnp.inf, probs)
    p2 = jnp.max(masked, axis=-1, keepdims=True)
    i2 = jnp.min(jnp.where(masked == p2, idx, E), axis=-1, keepdims=True)
    oh2 = (idx == i2)
    denom = p1 + p2 + 1e-8
    coef = (jnp.where(oh1, probs, 0.0) + jnp.where(oh2, probs, 0.0)) / denom

    acc = jax.lax.dot_general(
        coef, b2_ref[...], (((1,), (0,)), ((), ())),
        precision=jax.lax.Precision.HIGHEST,
        preferred_element_type=jnp.float32)
    xb = x.astype(jnp.bfloat16)
    cb = c.astype(jnp.bfloat16)
    for e in range(E):
        h = (jax.lax.dot_general(
                xb, w1h_ref[e], (((1,), (0,)), ((), ())),
                preferred_element_type=jnp.float32)
             + jax.lax.dot_general(
                cb, w1c_ref[e], (((1,), (0,)), ((), ())),
                preferred_element_type=jnp.float32)
             + b1_ref[e][None, :])
        h = (jnp.maximum(h, 0.0) * coef[:, e:e + 1]).astype(jnp.bfloat16)
        acc = acc + jax.lax.dot_general(
            h, w2_ref[e], (((1,), (0,)), ((), ())),
            preferred_element_type=jnp.float32)
    out_ref[...] = acc


def kernel(hidden_state, context, gate_W, gate_b, W1, b1, W2, b2):
    w1h = W1[:, :HIDDEN, :].astype(jnp.bfloat16)
    w1c = W1[:, HIDDEN:, :].astype(jnp.bfloat16)
    W2 = W2.astype(jnp.bfloat16)
    gb = gate_b.reshape(1, E)

    grid = (TOK // TILE,)
    out_shapes = (
        jax.ShapeDtypeStruct((TOK, OUT), jnp.float32),
        jax.ShapeDtypeStruct((TOK, E), jnp.float32),
    )
    full = lambda *dims: pl.BlockSpec(dims, lambda i: (0,) * len(dims))
    final_out, gate_probs = pl.pallas_call(
        _moe_tile_kernel,
        grid=grid,
        in_specs=[
            pl.BlockSpec((TILE, HIDDEN), lambda i: (i, 0)),
            pl.BlockSpec((TILE, CTX), lambda i: (i, 0)),
            full(HIDDEN, E),
            full(1, E),
            full(E, HIDDEN, HALF),
            full(E, CTX, HALF),
            full(E, HALF),
            full(E, HALF, OUT),
            full(E, OUT),
        ],
        out_specs=(
            pl.BlockSpec((TILE, OUT), lambda i: (i, 0)),
            pl.BlockSpec((TILE, E), lambda i: (i, 0)),
        ),
        out_shape=out_shapes,
        compiler_params=pltpu.CompilerParams(
            dimension_semantics=("arbitrary",),
        ),
    )(hidden_state, context, gate_W, gb, w1h, w1c, b1, W2, b2)
    return (final_out, gate_probs)


# speedup vs baseline: 4.3515x; 1.1714x over previous
import jax
import jax.numpy as jnp
from jax.experimental import pallas as pl
from jax.experimental.pallas import tpu as pltpu

TOK = 8192
HIDDEN = 768
OUT = 768
CTX = 64
E = 8
HALF = HIDDEN // 2
TILE = 256


def _moe_tile_kernel(x_ref, c_ref, gw_ref, gb_ref, w1h_ref, w1c_ref,
                     b1_ref, w2_ref, b2_ref, out_ref, probs_ref):
    x = x_ref[...]
    c = c_ref[...]

    logits = jax.lax.dot_general(
        x, gw_ref[...], (((1,), (0,)), ((), ())),
        precision=jax.lax.Precision.DEFAULT,
        preferred_element_type=jnp.float32) + gb_ref[...]
    m = jnp.max(logits, axis=-1, keepdims=True)
    ex = jnp.exp(logits - m)
    probs = ex / jnp.sum(ex, axis=-1, keepdims=True)
    probs_ref[...] = probs

    idx = jax.lax.broadcasted_iota(jnp.int32, probs.shape, 1)
    p1 = jnp.max(probs, axis=-1, keepdims=True)
    i1 = jnp.min(jnp.where(probs == p1, idx, E), axis=-1, keepdims=True)
    oh1 = (idx == i1)
    masked = jnp.where(oh1, -jnp.inf, probs)
    p2 = jnp.max(masked, axis=-1, keepdims=True)
    i2 = jnp.min(jnp.where(masked == p2, idx, E), axis=-1, keepdims=True)
    oh2 = (idx == i2)
    denom = p1 + p2 + 1e-8
    coef = (jnp.where(oh1, probs, 0.0) + jnp.where(oh2, probs, 0.0)) / denom

    xb = x.astype(jnp.bfloat16)
    cb = c.astype(jnp.bfloat16)
    h = (jax.lax.dot_general(
            xb, w1h_ref[...], (((1,), (0,)), ((), ())),
            preferred_element_type=jnp.float32)
         + jax.lax.dot_general(
            cb, w1c_ref[...], (((1,), (0,)), ((), ())),
            preferred_element_type=jnp.float32)
         + b1_ref[...])
    h = jnp.maximum(h, 0.0)
    h = h.reshape(TILE, E, HALF) * coef[:, :, None]
    hb = h.reshape(TILE, E * HALF).astype(jnp.bfloat16)
    acc = jax.lax.dot_general(
        hb, w2_ref[...], (((1,), (0,)), ((), ())),
        preferred_element_type=jnp.float32)
    acc = acc + jax.lax.dot_general(
        coef, b2_ref[...], (((1,), (0,)), ((), ())),
        precision=jax.lax.Precision.HIGHEST,
        preferred_element_type=jnp.float32)
    out_ref[...] = acc


def kernel(hidden_state, context, gate_W, gate_b, W1, b1, W2, b2):
    w1h = jnp.transpose(W1[:, :HIDDEN, :], (1, 0, 2)).reshape(
        HIDDEN, E * HALF).astype(jnp.bfloat16)
    w1c = jnp.transpose(W1[:, HIDDEN:, :], (1, 0, 2)).reshape(
        CTX, E * HALF).astype(jnp.bfloat16)
    b1f = b1.reshape(1, E * HALF)
    w2s = W2.reshape(E * HALF, OUT).astype(jnp.bfloat16)
    gb = gate_b.reshape(1, E)

    grid = (TOK // TILE,)
    out_shapes = (
        jax.ShapeDtypeStruct((TOK, OUT), jnp.float32),
        jax.ShapeDtypeStruct((TOK, E), jnp.float32),
    )
    full = lambda *dims: pl.BlockSpec(dims, lambda i: (0,) * len(dims))
    final_out, gate_probs = pl.pallas_call(
        _moe_tile_kernel,
        grid=grid,
        in_specs=[
            pl.BlockSpec((TILE, HIDDEN), lambda i: (i, 0)),
            pl.BlockSpec((TILE, CTX), lambda i: (i, 0)),
            full(HIDDEN, E),
            full(1, E),
            full(HIDDEN, E * HALF),
            full(CTX, E * HALF),
            full(1, E * HALF),
            full(E * HALF, OUT),
            full(E, OUT),
        ],
        out_specs=(
            pl.BlockSpec((TILE, OUT), lambda i: (i, 0)),
            pl.BlockSpec((TILE, E), lambda i: (i, 0)),
        ),
        out_shape=out_shapes,
        compiler_params=pltpu.CompilerParams(
            dimension_semantics=("parallel",),
        ),
    )(hidden_state, context, gate_W, gb, w1h, w1c, b1f, w2s, b2)
    return (final_out, gate_probs)


# speedup vs baseline: 5.2547x; 1.2076x over previous
import jax
import jax.numpy as jnp
from jax.experimental import pallas as pl
from jax.experimental.pallas import tpu as pltpu

TOK = 8192
HIDDEN = 768
OUT = 768
CTX = 64
E = 8
HALF = HIDDEN // 2
TILE = 256
CPAD = 128


def _moe_tile_kernel(x_ref, c_ref, gw_ref, gb_ref, w1h_ref, w1c_ref,
                     w2_ref, b2_ref, m_ref, out_ref, probs_ref):
    x = x_ref[...]
    c = c_ref[...]

    logits = jax.lax.dot_general(
        x, gw_ref[...], (((1,), (0,)), ((), ())),
        precision=jax.lax.Precision.DEFAULT,
        preferred_element_type=jnp.float32) + gb_ref[...]
    m = jnp.max(logits, axis=-1, keepdims=True)
    ex = jnp.exp(logits - m)
    probs = ex / jnp.sum(ex, axis=-1, keepdims=True)
    probs_ref[...] = probs

    idx = jax.lax.broadcasted_iota(jnp.int32, probs.shape, 1)
    p1 = jnp.max(probs, axis=-1, keepdims=True)
    i1 = jnp.min(jnp.where(probs == p1, idx, E), axis=-1, keepdims=True)
    oh1 = (idx == i1)
    masked = jnp.where(oh1, -jnp.inf, probs)
    p2 = jnp.max(masked, axis=-1, keepdims=True)
    i2 = jnp.min(jnp.where(masked == p2, idx, E), axis=-1, keepdims=True)
    oh2 = (idx == i2)
    denom = p1 + p2 + 1e-8
    coef = (jnp.where(oh1, probs, 0.0) + jnp.where(oh2, probs, 0.0)) / denom

    xb = x.astype(jnp.bfloat16)
    cb = c.astype(jnp.bfloat16)
    h = (jax.lax.dot_general(
            xb, w1h_ref[...], (((1,), (0,)), ((), ())),
            preferred_element_type=jnp.float32)
         + jax.lax.dot_general(
            cb, w1c_ref[...], (((1,), (0,)), ((), ())),
            preferred_element_type=jnp.float32))
    coefx = jax.lax.dot_general(
        coef.astype(jnp.bfloat16), m_ref[...], (((1,), (0,)), ((), ())),
        preferred_element_type=jnp.float32)
    hb = (jnp.maximum(h, 0.0) * coefx).astype(jnp.bfloat16)
    acc = jax.lax.dot_general(
        hb, w2_ref[...], (((1,), (0,)), ((), ())),
        preferred_element_type=jnp.float32)
    acc = acc + jax.lax.dot_general(
        coef, b2_ref[...], (((1,), (0,)), ((), ())),
        precision=jax.lax.Precision.DEFAULT,
        preferred_element_type=jnp.float32)
    out_ref[...] = acc


def kernel(hidden_state, context, gate_W, gate_b, W1, b1, W2, b2):
    w1h = jnp.transpose(W1[:, :HIDDEN, :], (1, 0, 2)).reshape(
        HIDDEN, E * HALF).astype(jnp.bfloat16)
    w1c_rows = jnp.transpose(W1[:, HIDDEN:, :], (1, 0, 2)).reshape(
        CTX, E * HALF)
    w1c = jnp.concatenate(
        [w1c_rows, b1.reshape(1, E * HALF),
         jnp.zeros((CPAD - CTX - 1, E * HALF), jnp.float32)],
        axis=0).astype(jnp.bfloat16)
    caug = jnp.concatenate(
        [context, jnp.ones((TOK, 1), jnp.float32),
         jnp.zeros((TOK, CPAD - CTX - 1), jnp.float32)], axis=1)
    w2s = W2.reshape(E * HALF, OUT).astype(jnp.bfloat16)
    gb = gate_b.reshape(1, E)
    mexp = (jnp.arange(E * HALF)[None, :] // HALF
            == jnp.arange(E)[:, None]).astype(jnp.bfloat16)

    grid = (TOK // TILE,)
    out_shapes = (
        jax.ShapeDtypeStruct((TOK, OUT), jnp.float32),
        jax.ShapeDtypeStruct((TOK, E), jnp.float32),
    )
    full = lambda *dims: pl.BlockSpec(dims, lambda i: (0,) * len(dims))
    final_out, gate_probs = pl.pallas_call(
        _moe_tile_kernel,
        grid=grid,
        in_specs=[
            pl.BlockSpec((TILE, HIDDEN), lambda i: (i, 0)),
            pl.BlockSpec((TILE, CPAD), lambda i: (i, 0)),
            full(HIDDEN, E),
            full(1, E),
            full(HIDDEN, E * HALF),
            full(CPAD, E * HALF),
            full(E * HALF, OUT),
            full(E, OUT),
            full(E, E * HALF),
        ],
        out_specs=(
            pl.BlockSpec((TILE, OUT), lambda i: (i, 0)),
            pl.BlockSpec((TILE, E), lambda i: (i, 0)),
        ),
        out_shape=out_shapes,
        compiler_params=pltpu.CompilerParams(
            dimension_semantics=("parallel",),
        ),
    )(hidden_state, caug, gate_W, gb, w1h, w1c, w2s, b2, mexp)
    return (final_out, gate_probs)


# speedup vs baseline: 5.4120x; 1.0299x over previous
import jax
import jax.numpy as jnp
from jax.experimental import pallas as pl
from jax.experimental.pallas import tpu as pltpu

TOK = 8192
HIDDEN = 768
OUT = 768
CTX = 64
E = 8
HALF = HIDDEN // 2
TILE = 512
CPAD = 128


def _moe_tile_kernel(x_ref, c_ref, gw_ref, gb_ref, w1h_ref, w1c_ref,
                     w2_ref, b2_ref, m_ref, out_ref, probs_ref):
    x = x_ref[...]
    c = c_ref[...]

    logits = jax.lax.dot_general(
        x, gw_ref[...], (((1,), (0,)), ((), ())),
        precision=jax.lax.Precision.DEFAULT,
        preferred_element_type=jnp.float32) + gb_ref[...]
    m = jnp.max(logits, axis=-1, keepdims=True)
    ex = jnp.exp(logits - m)
    probs = ex / jnp.sum(ex, axis=-1, keepdims=True)
    probs_ref[...] = probs

    idx = jax.lax.broadcasted_iota(jnp.int32, probs.shape, 1)
    p1 = jnp.max(probs, axis=-1, keepdims=True)
    i1 = jnp.min(jnp.where(probs == p1, idx, E), axis=-1, keepdims=True)
    oh1 = (idx == i1)
    masked = jnp.where(oh1, -jnp.inf, probs)
    p2 = jnp.max(masked, axis=-1, keepdims=True)
    i2 = jnp.min(jnp.where(masked == p2, idx, E), axis=-1, keepdims=True)
    oh2 = (idx == i2)
    denom = p1 + p2 + 1e-8
    coef = (jnp.where(oh1, probs, 0.0) + jnp.where(oh2, probs, 0.0)) / denom

    xb = x.astype(jnp.bfloat16)
    cb = c.astype(jnp.bfloat16)
    h = (jax.lax.dot_general(
            xb, w1h_ref[...], (((1,), (0,)), ((), ())),
            preferred_element_type=jnp.float32)
         + jax.lax.dot_general(
            cb, w1c_ref[...], (((1,), (0,)), ((), ())),
            preferred_element_type=jnp.float32))
    coefx = jax.lax.dot_general(
        coef.astype(jnp.bfloat16), m_ref[...], (((1,), (0,)), ((), ())),
        preferred_element_type=jnp.float32)
    hb = (jnp.maximum(h, 0.0) * coefx).astype(jnp.bfloat16)
    acc = jax.lax.dot_general(
        hb, w2_ref[...], (((1,), (0,)), ((), ())),
        preferred_element_type=jnp.float32)
    acc = acc + jax.lax.dot_general(
        coef, b2_ref[...], (((1,), (0,)), ((), ())),
        precision=jax.lax.Precision.DEFAULT,
        preferred_element_type=jnp.float32)
    out_ref[...] = acc


def kernel(hidden_state, context, gate_W, gate_b, W1, b1, W2, b2):
    w1h = jnp.transpose(W1[:, :HIDDEN, :], (1, 0, 2)).reshape(
        HIDDEN, E * HALF).astype(jnp.bfloat16)
    w1c_rows = jnp.transpose(W1[:, HIDDEN:, :], (1, 0, 2)).reshape(
        CTX, E * HALF)
    w1c = jnp.concatenate(
        [w1c_rows, b1.reshape(1, E * HALF),
         jnp.zeros((CPAD - CTX - 1, E * HALF), jnp.float32)],
        axis=0).astype(jnp.bfloat16)
    caug = jnp.concatenate(
        [context, jnp.ones((TOK, 1), jnp.float32),
         jnp.zeros((TOK, CPAD - CTX - 1), jnp.float32)], axis=1)
    w2s = W2.reshape(E * HALF, OUT).astype(jnp.bfloat16)
    gb = gate_b.reshape(1, E)
    mexp = (jnp.arange(E * HALF)[None, :] // HALF
            == jnp.arange(E)[:, None]).astype(jnp.bfloat16)

    grid = (TOK // TILE,)
    out_shapes = (
        jax.ShapeDtypeStruct((TOK, OUT), jnp.float32),
        jax.ShapeDtypeStruct((TOK, E), jnp.float32),
    )
    full = lambda *dims: pl.BlockSpec(dims, lambda i: (0,) * len(dims))
    final_out, gate_probs = pl.pallas_call(
        _moe_tile_kernel,
        grid=grid,
        in_specs=[
            pl.BlockSpec((TILE, HIDDEN), lambda i: (i, 0)),
            pl.BlockSpec((TILE, CPAD), lambda i: (i, 0)),
            full(HIDDEN, E),
            full(1, E),
            full(HIDDEN, E * HALF),
            full(CPAD, E * HALF),
            full(E * HALF, OUT),
            full(E, OUT),
            full(E, E * HALF),
        ],
        out_specs=(
            pl.BlockSpec((TILE, OUT), lambda i: (i, 0)),
            pl.BlockSpec((TILE, E), lambda i: (i, 0)),
        ),
        out_shape=out_shapes,
        compiler_params=pltpu.CompilerParams(
            dimension_semantics=("parallel",),
        ),
    )(hidden_state, caug, gate_W, gb, w1h, w1c, w2s, b2, mexp)
    return (final_out, gate_probs)
